# baseline (speedup 1.0000x reference)
import functools

import jax
import jax.numpy as jnp
from jax.experimental import pallas as pl
from jax.experimental.pallas import tpu as pltpu

BATCH = 1024
INPUT_DIM = 256
NUM_PROTOTYPES = 512
NUM_CLASSES = 8
PER_CLASS = NUM_PROTOTYPES // NUM_CLASSES
LAMBDA_VAL = 1.0


def _glmvq_kernel(x_ref, y_ref, p_ref, omega_hbm, out_ref, om_vmem, sem):
    copies = [pltpu.make_async_copy(omega_hbm.at[c], om_vmem.at[c], sem.at[c])
              for c in range(NUM_CLASSES)]
    for cp in copies:
        cp.start()
    x = x_ref[...]
    cols = []
    omega_sq = jnp.float32(0.0)
    for c in range(NUM_CLASSES):
        copies[c].wait()
        om = om_vmem[c]
        omega_sq = omega_sq + jnp.sum(om * om)
        tx = jax.lax.dot_general(
            x, om, (((1,), (1,)), ((), ())),
            preferred_element_type=jnp.float32)
        tp = jax.lax.dot_general(
            p_ref[:, c * INPUT_DIM:(c + 1) * INPUT_DIM], om,
            (((1,), (1,)), ((), ())),
            preferred_element_type=jnp.float32)
        norm_tx = jnp.sum(tx * tx, axis=1, keepdims=True)
        tpm2 = -2.0 * tp
        norm_tp = 0.25 * jnp.sum(tpm2 * tpm2, axis=1)
        crossm2 = jax.lax.dot_general(
            tx, tpm2, (((1,), (1,)), ((), ())),
            preferred_element_type=jnp.float32)
        q = crossm2 + norm_tp[None, :]
        cols.append(norm_tx + jnp.min(q, axis=1, keepdims=True))
    mind = jnp.concatenate(cols, axis=1)
    mt = mind.T
    y = y_ref[...]
    same = jax.lax.broadcasted_iota(jnp.int32, (NUM_CLASSES, BATCH), 0) == y
    inf = jnp.float32(jnp.inf)
    pos = jnp.min(jnp.where(same, mt, inf), axis=0)
    neg = jnp.min(jnp.where(same, inf, mt), axis=0)
    mu = (pos - neg) / (pos + neg)
    loss = jnp.mean(1.0 / (1.0 + jnp.exp(-LAMBDA_VAL * mu)))
    out_ref[...] = (loss + 0.01 * jnp.sqrt(omega_sq)).reshape(1, 1)


@functools.partial(jax.jit, static_argnames=())
def kernel(x, y, prototypes, omega):
    protos_r = prototypes.reshape(PER_CLASS, NUM_CLASSES * INPUT_DIM)
    y2 = y.reshape(1, BATCH)
    out = pl.pallas_call(
        _glmvq_kernel,
        out_shape=jax.ShapeDtypeStruct((1, 1), jnp.float32),
        in_specs=[
            pl.BlockSpec(memory_space=pltpu.MemorySpace.VMEM),
            pl.BlockSpec(memory_space=pltpu.MemorySpace.VMEM),
            pl.BlockSpec(memory_space=pltpu.MemorySpace.VMEM),
            pl.BlockSpec(memory_space=pltpu.MemorySpace.HBM),
        ],
        scratch_shapes=[
            pltpu.VMEM((NUM_CLASSES, INPUT_DIM, INPUT_DIM), jnp.float32),
            pltpu.SemaphoreType.DMA((NUM_CLASSES,)),
        ],
    )(x, y2, protos_r, omega)
    return out[0, 0]
